# baseline (speedup 1.0000x reference)
import functools
import jax
import jax.numpy as jnp
from jax import lax
from jax.experimental import pallas as pl
from jax.experimental.pallas import tpu as pltpu
from jax.experimental.pallas import tpu_sc as plsc

DIM = 1024
HIDDEN = 2816
HHALF = HIDDEN // 2
N_EXPERTS = 8
AUX_WEIGHT = 0.01

N = 4096
BG = 512
BM = 512
NM = N // BG
SMAX = 11776
NT = SMAX // BM

NW = 32
TPW = N // NW
CH_S = 32
NCH_S = TPW // CH_S
CH_G = 16
NCH_G = TPW // CH_G


def _top2(logits, n):
    iota = lax.broadcasted_iota(jnp.int32, (n, N_EXPERTS), 1)
    m1 = jnp.max(logits, axis=1, keepdims=True)
    i1 = jnp.min(jnp.where(logits == m1, iota, N_EXPERTS), axis=1,
                 keepdims=True)
    l2 = jnp.where(iota == i1, -jnp.inf, logits)
    m2 = jnp.max(l2, axis=1, keepdims=True)
    i2 = jnp.min(jnp.where(l2 == m2, iota, N_EXPERTS), axis=1,
                 keepdims=True)
    oh1 = (iota == i1).astype(jnp.float32)
    oh2 = (iota == i2).astype(jnp.float32)
    return m1, m2, oh1, oh2


def _gate_a_kernel(x_ref, gw_ref, cnt_ref, offp_ref, aux_ref,
                   psum_s, fsum_s, cnt_s):
    m = pl.program_id(0)
    n = x_ref.shape[0]
    logits = jnp.dot(x_ref[...], gw_ref[...].T,
                     preferred_element_type=jnp.float32)
    m1, m2, oh1, oh2 = _top2(logits, n)

    pe = jnp.exp(logits - m1)
    probs = pe / jnp.sum(pe, axis=1, keepdims=True)
    psum = jnp.sum(probs, axis=0, keepdims=True)
    fsum = jnp.sum(oh1, axis=0, keepdims=True)
    csum = jnp.sum(oh1 + oh2, axis=0, keepdims=True)

    @pl.when(m == 0)
    def _init():
        psum_s[...] = psum
        fsum_s[...] = fsum
        cnt_s[...] = csum

    @pl.when(m != 0)
    def _acc():
        psum_s[...] += psum
        fsum_s[...] += fsum
        cnt_s[...] += csum

    @pl.when(m == NM - 1)
    def _fin():
        padded = jnp.floor((cnt_s[...] + (BM - 1)) / BM) * BM
        ei = lax.broadcasted_iota(jnp.int32, (N_EXPERTS, N_EXPERTS), 0)
        ej = lax.broadcasted_iota(jnp.int32, (N_EXPERTS, N_EXPERTS), 1)
        slt = (ei < ej).astype(jnp.float32)
        offp = jnp.dot(padded, slt, preferred_element_type=jnp.float32)
        cnt_ref[...] = cnt_s[...].astype(jnp.int32)
        offp_ref[...] = offp.astype(jnp.int32)
        ntok = jnp.float32(N)
        aux_ref[...] = (AUX_WEIGHT * N_EXPERTS
                        * jnp.sum(psum_s[...] * fsum_s[...],
                                  keepdims=True).reshape(1, 1)
                        / (ntok * ntok))


def _gate_b_kernel(x_ref, gw_ref, offp_ref,
                   pos0_ref, pos1_ref, w0_ref, w1_ref, carry_s):
    m = pl.program_id(0)
    n = x_ref.shape[0]
    logits = jnp.dot(x_ref[...], gw_ref[...].T,
                     preferred_element_type=jnp.float32)
    m1, m2, oh1, oh2 = _top2(logits, n)

    t = jnp.exp(m2 - m1)
    w0_ref[...] = 1.0 / (1.0 + t)
    w1_ref[...] = t / (1.0 + t)

    @pl.when(m == 0)
    def _init():
        carry_s[...] = jnp.zeros_like(carry_s)

    o_all = jnp.concatenate([oh1, oh2], axis=0)
    ri = lax.broadcasted_iota(jnp.int32, (2 * n, 2 * n), 0)
    rj = lax.broadcasted_iota(jnp.int32, (2 * n, 2 * n), 1)
    tri = (ri >= rj).astype(jnp.float32)
    csum = jnp.dot(tri, o_all, preferred_element_type=jnp.float32)
    mtx = csum + carry_s[...] + offp_ref[...].astype(jnp.float32) - 1.0
    posall = jnp.sum(mtx * o_all, axis=1, keepdims=True)
    pos0_ref[...] = posall[:n].astype(jnp.int32)
    pos1_ref[...] = posall[n:].astype(jnp.int32)
    carry_s[...] += jnp.sum(o_all, axis=0, keepdims=True)


def _gate(x_flat, gate_w):
    cnt, offp, aux = pl.pallas_call(
        _gate_a_kernel,
        grid=(NM,),
        in_specs=[
            pl.BlockSpec((BG, DIM), lambda m: (m, 0)),
            pl.BlockSpec((N_EXPERTS, DIM), lambda m: (0, 0)),
        ],
        out_specs=[
            pl.BlockSpec((1, N_EXPERTS), lambda m: (0, 0)),
            pl.BlockSpec((1, N_EXPERTS), lambda m: (0, 0)),
            pl.BlockSpec((1, 1), lambda m: (0, 0)),
        ],
        out_shape=[
            jax.ShapeDtypeStruct((1, N_EXPERTS), jnp.int32),
            jax.ShapeDtypeStruct((1, N_EXPERTS), jnp.int32),
            jax.ShapeDtypeStruct((1, 1), jnp.float32),
        ],
        scratch_shapes=[pltpu.VMEM((1, N_EXPERTS), jnp.float32)] * 3,
    )(x_flat, gate_w)

    pos0, pos1, w0, w1 = pl.pallas_call(
        _gate_b_kernel,
        grid=(NM,),
        in_specs=[
            pl.BlockSpec((BG, DIM), lambda m: (m, 0)),
            pl.BlockSpec((N_EXPERTS, DIM), lambda m: (0, 0)),
            pl.BlockSpec((1, N_EXPERTS), lambda m: (0, 0)),
        ],
        out_specs=[
            pl.BlockSpec((BG, 1), lambda m: (m, 0)),
            pl.BlockSpec((BG, 1), lambda m: (m, 0)),
            pl.BlockSpec((BG, 1), lambda m: (m, 0)),
            pl.BlockSpec((BG, 1), lambda m: (m, 0)),
        ],
        out_shape=[
            jax.ShapeDtypeStruct((N, 1), jnp.int32),
            jax.ShapeDtypeStruct((N, 1), jnp.int32),
            jax.ShapeDtypeStruct((N, 1), jnp.float32),
            jax.ShapeDtypeStruct((N, 1), jnp.float32),
        ],
        scratch_shapes=[pltpu.VMEM((1, N_EXPERTS), jnp.float32)],
    )(x_flat, gate_w, offp)
    return pos0, pos1, w0, w1, cnt, offp, aux


_sc_mesh = plsc.VectorSubcoreMesh(core_axis_name="c", subcore_axis_name="s")


@functools.partial(
    pl.kernel,
    mesh=_sc_mesh,
    out_type=jax.ShapeDtypeStruct((SMAX, DIM), jnp.float32),
    scratch_types=[
        pltpu.VMEM((NCH_S, CH_S), jnp.int32),
        pltpu.VMEM((NCH_S, CH_S), jnp.int32),
        pltpu.VMEM((CH_S, DIM), jnp.float32),
        pltpu.SemaphoreType.DMA,
        pltpu.SemaphoreType.DMA,
    ],
)
def _sc_dispatch(x_hbm, pos0_hbm, pos1_hbm, xs_hbm,
                 p0_v, p1_v, xbuf, sem0, sem1):
    wid = lax.axis_index("s") * 2 + lax.axis_index("c")
    pltpu.sync_copy(pos0_hbm.at[wid], p0_v)
    pltpu.sync_copy(pos1_hbm.at[wid], p1_v)
    base = wid * TPW
    for c in range(NCH_S):
        pltpu.sync_copy(x_hbm.at[pl.ds(base + c * CH_S, CH_S)], xbuf)
        cp0 = pltpu.async_copy(xbuf, xs_hbm.at[p0_v.at[c]], sem0)
        cp1 = pltpu.async_copy(xbuf, xs_hbm.at[p1_v.at[c]], sem1)
        cp0.wait()
        cp1.wait()


def _ffn_kernel(h2, te_ref, tm_ref, rf_ref, par_ref, nxte_ref, hn_ref,
                xs_ref, w1_hbm, w3_hbm, w2_hbm, ysin_ref, ys_ref,
                w1b, w3b, w2b, sem1, sem3, sem2):
    i = pl.program_id(0)
    te = te_ref[i]
    p = par_ref[i]
    first = rf_ref[i] == 1

    def _w_copies(e, slot):
        return (
            pltpu.make_async_copy(
                w1_hbm.at[e, pl.ds(h2 * HHALF, HHALF), :],
                w1b.at[slot], sem1.at[slot]),
            pltpu.make_async_copy(
                w3_hbm.at[e, pl.ds(h2 * HHALF, HHALF), :],
                w3b.at[slot], sem3.at[slot]),
            pltpu.make_async_copy(
                w2_hbm.at[e, :, pl.ds(h2 * HHALF, HHALF)],
                w2b.at[slot], sem2.at[slot]),
        )

    def _start(e, slot):
        for c in _w_copies(e, slot):
            c.start()

    def _wait(e, slot):
        for c in _w_copies(e, slot):
            c.wait()

    @pl.when(i == 0)
    def _pro0():
        _start(te, 0)

    @pl.when((i == 0) & (hn_ref[i] == 1))
    def _pro1():
        _start(nxte_ref[i], 1)

    @pl.when(first & (i != 0) & (hn_ref[i] == 1) & (p == 0))
    def _fetch_n1():
        _start(nxte_ref[i], 1)

    @pl.when(first & (i != 0) & (hn_ref[i] == 1) & (p == 1))
    def _fetch_n0():
        _start(nxte_ref[i], 0)

    @pl.when(first & (p == 0))
    def _wait0():
        _wait(te, 0)

    @pl.when(first & (p == 1))
    def _wait1():
        _wait(te, 1)

    def _compute(slot):
        xl = xs_ref[...].astype(jnp.bfloat16)
        hb = (jax.nn.silu(jnp.dot(xl, w1b[slot].astype(jnp.bfloat16).T,
                                  preferred_element_type=jnp.float32))
              * jnp.dot(xl, w3b[slot].astype(jnp.bfloat16).T,
                        preferred_element_type=jnp.float32))
        contrib = jnp.dot(hb.astype(jnp.bfloat16),
                          w2b[slot].astype(jnp.bfloat16).T,
                          preferred_element_type=jnp.float32)
        if h2 == 0:
            ys_ref[...] = contrib
        else:
            ys_ref[...] = ysin_ref[...] + contrib

    @pl.when(p == 0)
    def _c0():
        _compute(0)

    @pl.when(p == 1)
    def _c1():
        _compute(1)


def _ffn(xs, W1, W3, W2, tile_e, tile_m, rf, par, nxte, hn):
    def half(h2, ysin):
        grid_spec = pltpu.PrefetchScalarGridSpec(
            num_scalar_prefetch=6,
            grid=(NT,),
            in_specs=[
                pl.BlockSpec((BM, DIM), lambda i, *pref: (pref[1][i], 0)),
                pl.BlockSpec(memory_space=pltpu.MemorySpace.HBM),
                pl.BlockSpec(memory_space=pltpu.MemorySpace.HBM),
                pl.BlockSpec(memory_space=pltpu.MemorySpace.HBM),
                (pl.BlockSpec((BM, DIM), lambda i, *pref: (pref[1][i], 0))
                 if h2 == 1 else
                 pl.BlockSpec((BM, DIM), lambda i, *pref: (0, 0))),
            ],
            out_specs=pl.BlockSpec((BM, DIM), lambda i, *pref: (pref[1][i], 0)),
            scratch_shapes=[
                pltpu.VMEM((2, HHALF, DIM), jnp.float32),
                pltpu.VMEM((2, HHALF, DIM), jnp.float32),
                pltpu.VMEM((2, DIM, HHALF), jnp.float32),
                pltpu.SemaphoreType.DMA((2,)),
                pltpu.SemaphoreType.DMA((2,)),
                pltpu.SemaphoreType.DMA((2,)),
            ],
        )
        return pl.pallas_call(
            functools.partial(_ffn_kernel, h2),
            grid_spec=grid_spec,
            out_shape=jax.ShapeDtypeStruct((SMAX, DIM), jnp.float32),
        )(tile_e, tile_m, rf, par, nxte, hn, xs, W1, W3, W2, ysin)

    ys0 = half(0, jnp.zeros((SMAX, DIM), jnp.float32))
    return half(1, ys0)


def tile_m_idx(pref, i):
    return pref[1][i]


def _tile_maps(cnt, offp):
    t_e = (cnt + BM - 1) // BM
    st = offp // BM
    ends = st + t_e
    nta = jnp.sum(t_e)
    i = jnp.arange(NT, dtype=jnp.int32)
    e_i = jnp.sum((i[:, None] >= ends[None, :]).astype(jnp.int32), axis=1)
    e_last = jnp.max(jnp.where(cnt > 0, jnp.arange(N_EXPERTS, dtype=jnp.int32),
                               0))
    tile_e = jnp.where(i < nta, jnp.minimum(e_i, N_EXPERTS - 1), e_last)
    tile_m = jnp.where(i < nta, i, NT - 1)
    return tile_e.astype(jnp.int32), tile_m.astype(jnp.int32)


def _ffn_sched(te):
    rf = jnp.concatenate([jnp.ones((1,), jnp.int32),
                          (te[1:] != te[:-1]).astype(jnp.int32)])
    ridx = jnp.cumsum(rf) - 1
    par = (ridx % 2).astype(jnp.int32)
    r = jnp.arange(NT, dtype=jnp.int32)
    r_onehot = (ridx[None, :] == r[:, None])
    start_pos = jnp.argmax(r_onehot, axis=1).astype(jnp.int32)
    run_e = te[start_pos]
    nrun = ridx[-1] + 1
    has = ((r + 1) < nrun).astype(jnp.int32)
    nxt_run_e = jnp.where((r + 1) < nrun,
                          run_e[jnp.minimum(r + 1, NT - 1)], run_e)
    nxte = nxt_run_e[ridx].astype(jnp.int32)
    hn = has[ridx].astype(jnp.int32)
    return rf.astype(jnp.int32), par, nxte, hn


@functools.partial(
    pl.kernel,
    mesh=_sc_mesh,
    out_type=jax.ShapeDtypeStruct((N, DIM), jnp.float32),
    scratch_types=(
        [pltpu.VMEM((NCH_G, CH_G), jnp.int32)] * 2
        + [pltpu.VMEM((TPW, 16), jnp.float32)] * 2
        + [pltpu.VMEM((CH_G, DIM), jnp.float32)] * 3
        + [pltpu.SemaphoreType.DMA] * 2
    ),
)
def _sc_combine(ys_hbm, ia_hbm, ic_hbm, wa_hbm, wc_hbm, out_hbm,
                iav, icv, wav, wcv, bufa, bufc, bufo, sa, sc):
    wid = lax.axis_index("s") * 2 + lax.axis_index("c")
    pltpu.sync_copy(ia_hbm.at[wid], iav)
    pltpu.sync_copy(ic_hbm.at[wid], icv)
    pltpu.sync_copy(wa_hbm.at[wid], wav)
    pltpu.sync_copy(wc_hbm.at[wid], wcv)
    base = wid * TPW
    for c in range(NCH_G):
        ca = pltpu.async_copy(ys_hbm.at[iav.at[c]], bufa, sa)
        cc = pltpu.async_copy(ys_hbm.at[icv.at[c]], bufc, sc)
        ca.wait()
        cc.wait()

        def _row(t, _):
            w0b = wav[c * CH_G + t, pl.ds(0, 16)]
            w1b = wcv[c * CH_G + t, pl.ds(0, 16)]
            for k in range(DIM // 16):
                va = bufa[t, pl.ds(16 * k, 16)]
                vc = bufc[t, pl.ds(16 * k, 16)]
                bufo[t, pl.ds(16 * k, 16)] = w0b * va + w1b * vc
            return 0

        lax.fori_loop(0, CH_G, _row, 0)
        pltpu.sync_copy(bufo, out_hbm.at[pl.ds(base + c * CH_G, CH_G)])


def kernel(x, gate_w, W1, W2, W3):
    B, T, C = x.shape
    x_flat = x.reshape(N, C)

    pos0, pos1, w0, w1, cnt, offp, aux = _gate(x_flat, gate_w)
    tile_e, tile_m = _tile_maps(cnt[0], offp[0])
    rf, par, nxte, hn = _ffn_sched(tile_e)

    p0r = pos0.reshape(NW, NCH_S, CH_S)
    p1r = pos1.reshape(NW, NCH_S, CH_S)
    xs = _sc_dispatch(x_flat, p0r, p1r)

    ys = _ffn(xs, W1, W3, W2, tile_e, tile_m, rf, par, nxte, hn)

    ia = pos0.reshape(NW, NCH_G, CH_G)
    ic = pos1.reshape(NW, NCH_G, CH_G)
    wa = jnp.broadcast_to(w0.reshape(NW, TPW, 1), (NW, TPW, 16))
    wc = jnp.broadcast_to(w1.reshape(NW, TPW, 1), (NW, TPW, 16))
    out = _sc_combine(ys, ia, ic, wa, wc)
    return out.reshape(B, T, C), aux.reshape(())[()]


# speedup vs baseline: 1.0147x; 1.0147x over previous
import functools
import jax
import jax.numpy as jnp
from jax import lax
from jax.experimental import pallas as pl
from jax.experimental.pallas import tpu as pltpu
from jax.experimental.pallas import tpu_sc as plsc

DIM = 1024
HIDDEN = 2816
HHALF = HIDDEN // 2
N_EXPERTS = 8
AUX_WEIGHT = 0.01

N = 4096
BG = 512
BM = 256
NM = N // BG
SMAX = 9984
NT = SMAX // BM

NW = 32
TPW = N // NW
CH_S = 32
NCH_S = TPW // CH_S
CH_G = 16
NCH_G = TPW // CH_G


def _top2(logits, n):
    iota = lax.broadcasted_iota(jnp.int32, (n, N_EXPERTS), 1)
    m1 = jnp.max(logits, axis=1, keepdims=True)
    i1 = jnp.min(jnp.where(logits == m1, iota, N_EXPERTS), axis=1,
                 keepdims=True)
    l2 = jnp.where(iota == i1, -jnp.inf, logits)
    m2 = jnp.max(l2, axis=1, keepdims=True)
    i2 = jnp.min(jnp.where(l2 == m2, iota, N_EXPERTS), axis=1,
                 keepdims=True)
    oh1 = (iota == i1).astype(jnp.float32)
    oh2 = (iota == i2).astype(jnp.float32)
    return m1, m2, oh1, oh2


def _gate_a_kernel(x_ref, gw_ref, cnt_ref, offp_ref, aux_ref,
                   psum_s, fsum_s, cnt_s):
    m = pl.program_id(0)
    n = x_ref.shape[0]
    logits = jnp.dot(x_ref[...], gw_ref[...].T,
                     preferred_element_type=jnp.float32)
    m1, m2, oh1, oh2 = _top2(logits, n)

    pe = jnp.exp(logits - m1)
    probs = pe / jnp.sum(pe, axis=1, keepdims=True)
    psum = jnp.sum(probs, axis=0, keepdims=True)
    fsum = jnp.sum(oh1, axis=0, keepdims=True)
    csum = jnp.sum(oh1 + oh2, axis=0, keepdims=True)

    @pl.when(m == 0)
    def _init():
        psum_s[...] = psum
        fsum_s[...] = fsum
        cnt_s[...] = csum

    @pl.when(m != 0)
    def _acc():
        psum_s[...] += psum
        fsum_s[...] += fsum
        cnt_s[...] += csum

    @pl.when(m == NM - 1)
    def _fin():
        padded = jnp.floor((cnt_s[...] + (BM - 1)) / BM) * BM
        ei = lax.broadcasted_iota(jnp.int32, (N_EXPERTS, N_EXPERTS), 0)
        ej = lax.broadcasted_iota(jnp.int32, (N_EXPERTS, N_EXPERTS), 1)
        slt = (ei < ej).astype(jnp.float32)
        offp = jnp.dot(padded, slt, preferred_element_type=jnp.float32)
        cnt_ref[...] = cnt_s[...].astype(jnp.int32)
        offp_ref[...] = offp.astype(jnp.int32)
        ntok = jnp.float32(N)
        aux_ref[...] = (AUX_WEIGHT * N_EXPERTS
                        * jnp.sum(psum_s[...] * fsum_s[...],
                                  keepdims=True).reshape(1, 1)
                        / (ntok * ntok))


def _gate_b_kernel(x_ref, gw_ref, offp_ref,
                   pos0_ref, pos1_ref, w0_ref, w1_ref, carry_s):
    m = pl.program_id(0)
    n = x_ref.shape[0]
    logits = jnp.dot(x_ref[...], gw_ref[...].T,
                     preferred_element_type=jnp.float32)
    m1, m2, oh1, oh2 = _top2(logits, n)

    t = jnp.exp(m2 - m1)
    w0_ref[...] = 1.0 / (1.0 + t)
    w1_ref[...] = t / (1.0 + t)

    @pl.when(m == 0)
    def _init():
        carry_s[...] = jnp.zeros_like(carry_s)

    o_all = jnp.concatenate([oh1, oh2], axis=0)
    ri = lax.broadcasted_iota(jnp.int32, (2 * n, 2 * n), 0)
    rj = lax.broadcasted_iota(jnp.int32, (2 * n, 2 * n), 1)
    tri = (ri >= rj).astype(jnp.float32)
    csum = jnp.dot(tri, o_all, preferred_element_type=jnp.float32)
    mtx = csum + carry_s[...] + offp_ref[...].astype(jnp.float32) - 1.0
    posall = jnp.sum(mtx * o_all, axis=1, keepdims=True)
    pos0_ref[...] = posall[:n].astype(jnp.int32)
    pos1_ref[...] = posall[n:].astype(jnp.int32)
    carry_s[...] += jnp.sum(o_all, axis=0, keepdims=True)


def _gate(x_flat, gate_w):
    cnt, offp, aux = pl.pallas_call(
        _gate_a_kernel,
        grid=(NM,),
        in_specs=[
            pl.BlockSpec((BG, DIM), lambda m: (m, 0)),
            pl.BlockSpec((N_EXPERTS, DIM), lambda m: (0, 0)),
        ],
        out_specs=[
            pl.BlockSpec((1, N_EXPERTS), lambda m: (0, 0)),
            pl.BlockSpec((1, N_EXPERTS), lambda m: (0, 0)),
            pl.BlockSpec((1, 1), lambda m: (0, 0)),
        ],
        out_shape=[
            jax.ShapeDtypeStruct((1, N_EXPERTS), jnp.int32),
            jax.ShapeDtypeStruct((1, N_EXPERTS), jnp.int32),
            jax.ShapeDtypeStruct((1, 1), jnp.float32),
        ],
        scratch_shapes=[pltpu.VMEM((1, N_EXPERTS), jnp.float32)] * 3,
    )(x_flat, gate_w)

    pos0, pos1, w0, w1 = pl.pallas_call(
        _gate_b_kernel,
        grid=(NM,),
        in_specs=[
            pl.BlockSpec((BG, DIM), lambda m: (m, 0)),
            pl.BlockSpec((N_EXPERTS, DIM), lambda m: (0, 0)),
            pl.BlockSpec((1, N_EXPERTS), lambda m: (0, 0)),
        ],
        out_specs=[
            pl.BlockSpec((BG, 1), lambda m: (m, 0)),
            pl.BlockSpec((BG, 1), lambda m: (m, 0)),
            pl.BlockSpec((BG, 1), lambda m: (m, 0)),
            pl.BlockSpec((BG, 1), lambda m: (m, 0)),
        ],
        out_shape=[
            jax.ShapeDtypeStruct((N, 1), jnp.int32),
            jax.ShapeDtypeStruct((N, 1), jnp.int32),
            jax.ShapeDtypeStruct((N, 1), jnp.float32),
            jax.ShapeDtypeStruct((N, 1), jnp.float32),
        ],
        scratch_shapes=[pltpu.VMEM((1, N_EXPERTS), jnp.float32)],
    )(x_flat, gate_w, offp)
    return pos0, pos1, w0, w1, cnt, offp, aux


_sc_mesh = plsc.VectorSubcoreMesh(core_axis_name="c", subcore_axis_name="s")


@functools.partial(
    pl.kernel,
    mesh=_sc_mesh,
    out_type=jax.ShapeDtypeStruct((SMAX, DIM), jnp.float32),
    scratch_types=[
        pltpu.VMEM((NCH_S, CH_S), jnp.int32),
        pltpu.VMEM((NCH_S, CH_S), jnp.int32),
        pltpu.VMEM((CH_S, DIM), jnp.float32),
        pltpu.SemaphoreType.DMA,
        pltpu.SemaphoreType.DMA,
    ],
)
def _sc_dispatch(x_hbm, pos0_hbm, pos1_hbm, xs_hbm,
                 p0_v, p1_v, xbuf, sem0, sem1):
    wid = lax.axis_index("s") * 2 + lax.axis_index("c")
    pltpu.sync_copy(pos0_hbm.at[wid], p0_v)
    pltpu.sync_copy(pos1_hbm.at[wid], p1_v)
    base = wid * TPW
    for c in range(NCH_S):
        pltpu.sync_copy(x_hbm.at[pl.ds(base + c * CH_S, CH_S)], xbuf)
        cp0 = pltpu.async_copy(xbuf, xs_hbm.at[p0_v.at[c]], sem0)
        cp1 = pltpu.async_copy(xbuf, xs_hbm.at[p1_v.at[c]], sem1)
        cp0.wait()
        cp1.wait()


def _ffn_kernel(h2, te_ref, tm_ref, rf_ref, par_ref, nxte_ref, hn_ref,
                xs_ref, w1_hbm, w3_hbm, w2_hbm, ysin_ref, ys_ref,
                w1b, w3b, w2b, sem1, sem3, sem2):
    i = pl.program_id(0)
    te = te_ref[i]
    p = par_ref[i]
    first = rf_ref[i] == 1

    def _w_copies(e, slot):
        return (
            pltpu.make_async_copy(
                w1_hbm.at[e, pl.ds(h2 * HHALF, HHALF), :],
                w1b.at[slot], sem1.at[slot]),
            pltpu.make_async_copy(
                w3_hbm.at[e, pl.ds(h2 * HHALF, HHALF), :],
                w3b.at[slot], sem3.at[slot]),
            pltpu.make_async_copy(
                w2_hbm.at[e, :, pl.ds(h2 * HHALF, HHALF)],
                w2b.at[slot], sem2.at[slot]),
        )

    def _start(e, slot):
        for c in _w_copies(e, slot):
            c.start()

    def _wait(e, slot):
        for c in _w_copies(e, slot):
            c.wait()

    @pl.when(i == 0)
    def _pro0():
        _start(te, 0)

    @pl.when((i == 0) & (hn_ref[i] == 1))
    def _pro1():
        _start(nxte_ref[i], 1)

    @pl.when(first & (i != 0) & (hn_ref[i] == 1) & (p == 0))
    def _fetch_n1():
        _start(nxte_ref[i], 1)

    @pl.when(first & (i != 0) & (hn_ref[i] == 1) & (p == 1))
    def _fetch_n0():
        _start(nxte_ref[i], 0)

    @pl.when(first & (p == 0))
    def _wait0():
        _wait(te, 0)

    @pl.when(first & (p == 1))
    def _wait1():
        _wait(te, 1)

    def _compute(slot):
        xl = xs_ref[...].astype(jnp.bfloat16)
        hb = (jax.nn.silu(jnp.dot(xl, w1b[slot].astype(jnp.bfloat16).T,
                                  preferred_element_type=jnp.float32))
              * jnp.dot(xl, w3b[slot].astype(jnp.bfloat16).T,
                        preferred_element_type=jnp.float32))
        contrib = jnp.dot(hb.astype(jnp.bfloat16),
                          w2b[slot].astype(jnp.bfloat16).T,
                          preferred_element_type=jnp.float32)
        if h2 == 0:
            ys_ref[...] = contrib
        else:
            ys_ref[...] = ysin_ref[...] + contrib

    @pl.when(p == 0)
    def _c0():
        _compute(0)

    @pl.when(p == 1)
    def _c1():
        _compute(1)


def _ffn(xs, W1, W3, W2, tile_e, tile_m, rf, par, nxte, hn):
    def half(h2, ysin):
        grid_spec = pltpu.PrefetchScalarGridSpec(
            num_scalar_prefetch=6,
            grid=(NT,),
            in_specs=[
                pl.BlockSpec((BM, DIM), lambda i, *pref: (pref[1][i], 0)),
                pl.BlockSpec(memory_space=pltpu.MemorySpace.HBM),
                pl.BlockSpec(memory_space=pltpu.MemorySpace.HBM),
                pl.BlockSpec(memory_space=pltpu.MemorySpace.HBM),
                (pl.BlockSpec((BM, DIM), lambda i, *pref: (pref[1][i], 0))
                 if h2 == 1 else
                 pl.BlockSpec((BM, DIM), lambda i, *pref: (0, 0))),
            ],
            out_specs=pl.BlockSpec((BM, DIM), lambda i, *pref: (pref[1][i], 0)),
            scratch_shapes=[
                pltpu.VMEM((2, HHALF, DIM), jnp.float32),
                pltpu.VMEM((2, HHALF, DIM), jnp.float32),
                pltpu.VMEM((2, DIM, HHALF), jnp.float32),
                pltpu.SemaphoreType.DMA((2,)),
                pltpu.SemaphoreType.DMA((2,)),
                pltpu.SemaphoreType.DMA((2,)),
            ],
        )
        return pl.pallas_call(
            functools.partial(_ffn_kernel, h2),
            grid_spec=grid_spec,
            out_shape=jax.ShapeDtypeStruct((SMAX, DIM), jnp.float32),
        )(tile_e, tile_m, rf, par, nxte, hn, xs, W1, W3, W2, ysin)

    ys0 = half(0, jnp.zeros((SMAX, DIM), jnp.float32))
    return half(1, ys0)


def tile_m_idx(pref, i):
    return pref[1][i]


def _tile_maps(cnt, offp):
    t_e = (cnt + BM - 1) // BM
    st = offp // BM
    ends = st + t_e
    nta = jnp.sum(t_e)
    i = jnp.arange(NT, dtype=jnp.int32)
    e_i = jnp.sum((i[:, None] >= ends[None, :]).astype(jnp.int32), axis=1)
    e_last = jnp.max(jnp.where(cnt > 0, jnp.arange(N_EXPERTS, dtype=jnp.int32),
                               0))
    tile_e = jnp.where(i < nta, jnp.minimum(e_i, N_EXPERTS - 1), e_last)
    tile_m = jnp.where(i < nta, i, NT - 1)
    return tile_e.astype(jnp.int32), tile_m.astype(jnp.int32)


def _ffn_sched(te):
    rf = jnp.concatenate([jnp.ones((1,), jnp.int32),
                          (te[1:] != te[:-1]).astype(jnp.int32)])
    ridx = jnp.cumsum(rf) - 1
    par = (ridx % 2).astype(jnp.int32)
    r = jnp.arange(NT, dtype=jnp.int32)
    r_onehot = (ridx[None, :] == r[:, None])
    start_pos = jnp.argmax(r_onehot, axis=1).astype(jnp.int32)
    run_e = te[start_pos]
    nrun = ridx[-1] + 1
    has = ((r + 1) < nrun).astype(jnp.int32)
    nxt_run_e = jnp.where((r + 1) < nrun,
                          run_e[jnp.minimum(r + 1, NT - 1)], run_e)
    nxte = nxt_run_e[ridx].astype(jnp.int32)
    hn = has[ridx].astype(jnp.int32)
    return rf.astype(jnp.int32), par, nxte, hn


@functools.partial(
    pl.kernel,
    mesh=_sc_mesh,
    out_type=jax.ShapeDtypeStruct((N, DIM), jnp.float32),
    scratch_types=(
        [pltpu.VMEM((NCH_G, CH_G), jnp.int32)] * 2
        + [pltpu.VMEM((TPW, 16), jnp.float32)] * 2
        + [pltpu.VMEM((CH_G, DIM), jnp.float32)] * 3
        + [pltpu.SemaphoreType.DMA] * 2
    ),
)
def _sc_combine(ys_hbm, ia_hbm, ic_hbm, wa_hbm, wc_hbm, out_hbm,
                iav, icv, wav, wcv, bufa, bufc, bufo, sa, sc):
    wid = lax.axis_index("s") * 2 + lax.axis_index("c")
    pltpu.sync_copy(ia_hbm.at[wid], iav)
    pltpu.sync_copy(ic_hbm.at[wid], icv)
    pltpu.sync_copy(wa_hbm.at[wid], wav)
    pltpu.sync_copy(wc_hbm.at[wid], wcv)
    base = wid * TPW
    for c in range(NCH_G):
        ca = pltpu.async_copy(ys_hbm.at[iav.at[c]], bufa, sa)
        cc = pltpu.async_copy(ys_hbm.at[icv.at[c]], bufc, sc)
        ca.wait()
        cc.wait()

        def _row(t, _):
            w0b = wav[c * CH_G + t, pl.ds(0, 16)]
            w1b = wcv[c * CH_G + t, pl.ds(0, 16)]
            for k in range(DIM // 16):
                va = bufa[t, pl.ds(16 * k, 16)]
                vc = bufc[t, pl.ds(16 * k, 16)]
                bufo[t, pl.ds(16 * k, 16)] = w0b * va + w1b * vc
            return 0

        lax.fori_loop(0, CH_G, _row, 0)
        pltpu.sync_copy(bufo, out_hbm.at[pl.ds(base + c * CH_G, CH_G)])


def kernel(x, gate_w, W1, W2, W3):
    B, T, C = x.shape
    x_flat = x.reshape(N, C)

    pos0, pos1, w0, w1, cnt, offp, aux = _gate(x_flat, gate_w)
    tile_e, tile_m = _tile_maps(cnt[0], offp[0])
    rf, par, nxte, hn = _ffn_sched(tile_e)

    p0r = pos0.reshape(NW, NCH_S, CH_S)
    p1r = pos1.reshape(NW, NCH_S, CH_S)
    xs = _sc_dispatch(x_flat, p0r, p1r)

    ys = _ffn(xs, W1, W3, W2, tile_e, tile_m, rf, par, nxte, hn)

    ia = pos0.reshape(NW, NCH_G, CH_G)
    ic = pos1.reshape(NW, NCH_G, CH_G)
    wa = jnp.broadcast_to(w0.reshape(NW, TPW, 1), (NW, TPW, 16))
    wc = jnp.broadcast_to(w1.reshape(NW, TPW, 1), (NW, TPW, 16))
    out = _sc_combine(ys, ia, ic, wa, wc)
    return out.reshape(B, T, C), aux.reshape(())[()]


# speedup vs baseline: 1.0497x; 1.0346x over previous
import functools
import jax
import jax.numpy as jnp
from jax import lax
from jax.experimental import pallas as pl
from jax.experimental.pallas import tpu as pltpu
from jax.experimental.pallas import tpu_sc as plsc

DIM = 1024
HIDDEN = 2816
HHALF = HIDDEN // 2
N_EXPERTS = 8
AUX_WEIGHT = 0.01

N = 4096
BG = 512
BM = 256
NM = N // BG
SMAX = 9984
NT = SMAX // BM

NW = 32
TPW = N // NW
CH_S = 32
NCH_S = TPW // CH_S
CH_G = 16
NCH_G = TPW // CH_G


def _top2(logits, n):
    iota = lax.broadcasted_iota(jnp.int32, (n, N_EXPERTS), 1)
    m1 = jnp.max(logits, axis=1, keepdims=True)
    i1 = jnp.min(jnp.where(logits == m1, iota, N_EXPERTS), axis=1,
                 keepdims=True)
    l2 = jnp.where(iota == i1, -jnp.inf, logits)
    m2 = jnp.max(l2, axis=1, keepdims=True)
    i2 = jnp.min(jnp.where(l2 == m2, iota, N_EXPERTS), axis=1,
                 keepdims=True)
    oh1 = (iota == i1).astype(jnp.float32)
    oh2 = (iota == i2).astype(jnp.float32)
    return m1, m2, oh1, oh2


def _gate_a_kernel(x_ref, gw_ref, cnt_ref, offp_ref, aux_ref,
                   psum_s, fsum_s, cnt_s):
    m = pl.program_id(0)
    n = x_ref.shape[0]
    logits = jnp.dot(x_ref[...], gw_ref[...].T,
                     preferred_element_type=jnp.float32)
    m1, m2, oh1, oh2 = _top2(logits, n)

    pe = jnp.exp(logits - m1)
    probs = pe / jnp.sum(pe, axis=1, keepdims=True)
    psum = jnp.sum(probs, axis=0, keepdims=True)
    fsum = jnp.sum(oh1, axis=0, keepdims=True)
    csum = jnp.sum(oh1 + oh2, axis=0, keepdims=True)

    @pl.when(m == 0)
    def _init():
        psum_s[...] = psum
        fsum_s[...] = fsum
        cnt_s[...] = csum

    @pl.when(m != 0)
    def _acc():
        psum_s[...] += psum
        fsum_s[...] += fsum
        cnt_s[...] += csum

    @pl.when(m == NM - 1)
    def _fin():
        padded = jnp.floor((cnt_s[...] + (BM - 1)) / BM) * BM
        ei = lax.broadcasted_iota(jnp.int32, (N_EXPERTS, N_EXPERTS), 0)
        ej = lax.broadcasted_iota(jnp.int32, (N_EXPERTS, N_EXPERTS), 1)
        slt = (ei < ej).astype(jnp.float32)
        offp = jnp.dot(padded, slt, preferred_element_type=jnp.float32)
        cnt_ref[...] = cnt_s[...].astype(jnp.int32)
        offp_ref[...] = offp.astype(jnp.int32)
        ntok = jnp.float32(N)
        aux_ref[...] = (AUX_WEIGHT * N_EXPERTS
                        * jnp.sum(psum_s[...] * fsum_s[...],
                                  keepdims=True).reshape(1, 1)
                        / (ntok * ntok))


def _gate_b_kernel(x_ref, gw_ref, offp_ref,
                   pos0_ref, pos1_ref, w0_ref, w1_ref, carry_s):
    m = pl.program_id(0)
    n = x_ref.shape[0]
    logits = jnp.dot(x_ref[...], gw_ref[...].T,
                     preferred_element_type=jnp.float32)
    m1, m2, oh1, oh2 = _top2(logits, n)

    t = jnp.exp(m2 - m1)
    w0_ref[...] = 1.0 / (1.0 + t)
    w1_ref[...] = t / (1.0 + t)

    @pl.when(m == 0)
    def _init():
        carry_s[...] = jnp.zeros_like(carry_s)

    o_all = jnp.concatenate([oh1, oh2], axis=0)
    ri = lax.broadcasted_iota(jnp.int32, (2 * n, 2 * n), 0)
    rj = lax.broadcasted_iota(jnp.int32, (2 * n, 2 * n), 1)
    tri = (ri >= rj).astype(jnp.bfloat16)
    csum = jnp.dot(tri, o_all.astype(jnp.bfloat16),
                   preferred_element_type=jnp.float32)
    mtx = csum + carry_s[...] + offp_ref[...].astype(jnp.float32) - 1.0
    posall = jnp.sum(mtx * o_all, axis=1, keepdims=True)
    pos0_ref[...] = posall[:n].astype(jnp.int32)
    pos1_ref[...] = posall[n:].astype(jnp.int32)
    carry_s[...] += jnp.sum(o_all, axis=0, keepdims=True)


def _gate(x_flat, gate_w):
    cnt, offp, aux = pl.pallas_call(
        _gate_a_kernel,
        grid=(NM,),
        in_specs=[
            pl.BlockSpec((BG, DIM), lambda m: (m, 0)),
            pl.BlockSpec((N_EXPERTS, DIM), lambda m: (0, 0)),
        ],
        out_specs=[
            pl.BlockSpec((1, N_EXPERTS), lambda m: (0, 0)),
            pl.BlockSpec((1, N_EXPERTS), lambda m: (0, 0)),
            pl.BlockSpec((1, 1), lambda m: (0, 0)),
        ],
        out_shape=[
            jax.ShapeDtypeStruct((1, N_EXPERTS), jnp.int32),
            jax.ShapeDtypeStruct((1, N_EXPERTS), jnp.int32),
            jax.ShapeDtypeStruct((1, 1), jnp.float32),
        ],
        scratch_shapes=[pltpu.VMEM((1, N_EXPERTS), jnp.float32)] * 3,
    )(x_flat, gate_w)

    pos0, pos1, w0, w1 = pl.pallas_call(
        _gate_b_kernel,
        grid=(NM,),
        in_specs=[
            pl.BlockSpec((BG, DIM), lambda m: (m, 0)),
            pl.BlockSpec((N_EXPERTS, DIM), lambda m: (0, 0)),
            pl.BlockSpec((1, N_EXPERTS), lambda m: (0, 0)),
        ],
        out_specs=[
            pl.BlockSpec((BG, 1), lambda m: (m, 0)),
            pl.BlockSpec((BG, 1), lambda m: (m, 0)),
            pl.BlockSpec((BG, 1), lambda m: (m, 0)),
            pl.BlockSpec((BG, 1), lambda m: (m, 0)),
        ],
        out_shape=[
            jax.ShapeDtypeStruct((N, 1), jnp.int32),
            jax.ShapeDtypeStruct((N, 1), jnp.int32),
            jax.ShapeDtypeStruct((N, 1), jnp.float32),
            jax.ShapeDtypeStruct((N, 1), jnp.float32),
        ],
        scratch_shapes=[pltpu.VMEM((1, N_EXPERTS), jnp.float32)],
    )(x_flat, gate_w, offp)
    return pos0, pos1, w0, w1, cnt, offp, aux


_sc_mesh = plsc.VectorSubcoreMesh(core_axis_name="c", subcore_axis_name="s")


@functools.partial(
    pl.kernel,
    mesh=_sc_mesh,
    out_type=jax.ShapeDtypeStruct((SMAX, DIM), jnp.float32),
    scratch_types=[
        pltpu.VMEM((NCH_S, CH_S), jnp.int32),
        pltpu.VMEM((NCH_S, CH_S), jnp.int32),
        pltpu.VMEM((CH_S, DIM), jnp.float32),
        pltpu.SemaphoreType.DMA,
        pltpu.SemaphoreType.DMA,
    ],
)
def _sc_dispatch(x_hbm, pos0_hbm, pos1_hbm, xs_hbm,
                 p0_v, p1_v, xbuf, sem0, sem1):
    wid = lax.axis_index("s") * 2 + lax.axis_index("c")
    pltpu.sync_copy(pos0_hbm.at[wid], p0_v)
    pltpu.sync_copy(pos1_hbm.at[wid], p1_v)
    base = wid * TPW
    for c in range(NCH_S):
        pltpu.sync_copy(x_hbm.at[pl.ds(base + c * CH_S, CH_S)], xbuf)
        cp0 = pltpu.async_copy(xbuf, xs_hbm.at[p0_v.at[c]], sem0)
        cp1 = pltpu.async_copy(xbuf, xs_hbm.at[p1_v.at[c]], sem1)
        cp0.wait()
        cp1.wait()


def _ffn_kernel(h2, te_ref, tm_ref, rf_ref, par_ref, nxte_ref, hn_ref,
                xs_ref, w1_hbm, w3_hbm, w2_hbm, ysin_ref, ys_ref,
                w1b, w3b, w2b, sem1, sem3, sem2):
    i = pl.program_id(0)
    te = te_ref[i]
    p = par_ref[i]
    first = rf_ref[i] == 1

    def _w_copies(e, slot):
        return (
            pltpu.make_async_copy(
                w1_hbm.at[e, pl.ds(h2 * HHALF, HHALF), :],
                w1b.at[slot], sem1.at[slot]),
            pltpu.make_async_copy(
                w3_hbm.at[e, pl.ds(h2 * HHALF, HHALF), :],
                w3b.at[slot], sem3.at[slot]),
            pltpu.make_async_copy(
                w2_hbm.at[e, :, pl.ds(h2 * HHALF, HHALF)],
                w2b.at[slot], sem2.at[slot]),
        )

    def _start(e, slot):
        for c in _w_copies(e, slot):
            c.start()

    def _wait(e, slot):
        for c in _w_copies(e, slot):
            c.wait()

    @pl.when(i == 0)
    def _pro0():
        _start(te, 0)

    @pl.when((i == 0) & (hn_ref[i] == 1))
    def _pro1():
        _start(nxte_ref[i], 1)

    @pl.when(first & (i != 0) & (hn_ref[i] == 1) & (p == 0))
    def _fetch_n1():
        _start(nxte_ref[i], 1)

    @pl.when(first & (i != 0) & (hn_ref[i] == 1) & (p == 1))
    def _fetch_n0():
        _start(nxte_ref[i], 0)

    @pl.when(first & (p == 0))
    def _wait0():
        _wait(te, 0)

    @pl.when(first & (p == 1))
    def _wait1():
        _wait(te, 1)

    def _compute(slot):
        xl = xs_ref[...].astype(jnp.bfloat16)
        hb = (jax.nn.silu(jnp.dot(xl, w1b[slot].astype(jnp.bfloat16).T,
                                  preferred_element_type=jnp.float32))
              * jnp.dot(xl, w3b[slot].astype(jnp.bfloat16).T,
                        preferred_element_type=jnp.float32))
        contrib = jnp.dot(hb.astype(jnp.bfloat16),
                          w2b[slot].astype(jnp.bfloat16).T,
                          preferred_element_type=jnp.float32)
        if h2 == 0:
            ys_ref[...] = contrib
        else:
            ys_ref[...] = ysin_ref[...] + contrib

    @pl.when(p == 0)
    def _c0():
        _compute(0)

    @pl.when(p == 1)
    def _c1():
        _compute(1)


def _ffn(xs, W1, W3, W2, tile_e, tile_m, rf, par, nxte, hn):
    def half(h2, ysin):
        grid_spec = pltpu.PrefetchScalarGridSpec(
            num_scalar_prefetch=6,
            grid=(NT,),
            in_specs=[
                pl.BlockSpec((BM, DIM), lambda i, *pref: (pref[1][i], 0)),
                pl.BlockSpec(memory_space=pltpu.MemorySpace.HBM),
                pl.BlockSpec(memory_space=pltpu.MemorySpace.HBM),
                pl.BlockSpec(memory_space=pltpu.MemorySpace.HBM),
                (pl.BlockSpec((BM, DIM), lambda i, *pref: (pref[1][i], 0))
                 if h2 == 1 else
                 pl.BlockSpec((BM, DIM), lambda i, *pref: (0, 0))),
            ],
            out_specs=pl.BlockSpec((BM, DIM), lambda i, *pref: (pref[1][i], 0)),
            scratch_shapes=[
                pltpu.VMEM((2, HHALF, DIM), jnp.float32),
                pltpu.VMEM((2, HHALF, DIM), jnp.float32),
                pltpu.VMEM((2, DIM, HHALF), jnp.float32),
                pltpu.SemaphoreType.DMA((2,)),
                pltpu.SemaphoreType.DMA((2,)),
                pltpu.SemaphoreType.DMA((2,)),
            ],
        )
        return pl.pallas_call(
            functools.partial(_ffn_kernel, h2),
            grid_spec=grid_spec,
            out_shape=jax.ShapeDtypeStruct((SMAX, DIM), jnp.float32),
        )(tile_e, tile_m, rf, par, nxte, hn, xs, W1, W3, W2, ysin)

    ys0 = half(0, xs)
    return half(1, ys0)


def tile_m_idx(pref, i):
    return pref[1][i]


def _tile_maps(cnt, offp):
    t_e = (cnt + BM - 1) // BM
    st = offp // BM
    ends = st + t_e
    nta = jnp.sum(t_e)
    i = jnp.arange(NT, dtype=jnp.int32)
    e_i = jnp.sum((i[:, None] >= ends[None, :]).astype(jnp.int32), axis=1)
    e_last = jnp.max(jnp.where(cnt > 0, jnp.arange(N_EXPERTS, dtype=jnp.int32),
                               0))
    tile_e = jnp.where(i < nta, jnp.minimum(e_i, N_EXPERTS - 1), e_last)
    tile_m = jnp.where(i < nta, i, NT - 1)
    return tile_e.astype(jnp.int32), tile_m.astype(jnp.int32)


def _ffn_sched(te):
    rf = jnp.concatenate([jnp.ones((1,), jnp.int32),
                          (te[1:] != te[:-1]).astype(jnp.int32)])
    ridx = jnp.cumsum(rf) - 1
    par = (ridx % 2).astype(jnp.int32)
    r = jnp.arange(NT, dtype=jnp.int32)
    r_onehot = (ridx[None, :] == r[:, None])
    start_pos = jnp.argmax(r_onehot, axis=1).astype(jnp.int32)
    run_e = te[start_pos]
    nrun = ridx[-1] + 1
    has = ((r + 1) < nrun).astype(jnp.int32)
    nxt_run_e = jnp.where((r + 1) < nrun,
                          run_e[jnp.minimum(r + 1, NT - 1)], run_e)
    nxte = nxt_run_e[ridx].astype(jnp.int32)
    hn = has[ridx].astype(jnp.int32)
    return rf.astype(jnp.int32), par, nxte, hn


@functools.partial(
    pl.kernel,
    mesh=_sc_mesh,
    out_type=jax.ShapeDtypeStruct((N, DIM), jnp.float32),
    scratch_types=(
        [pltpu.VMEM((NCH_G, CH_G), jnp.int32)] * 2
        + [pltpu.VMEM((TPW, 16), jnp.float32)] * 2
        + [pltpu.VMEM((CH_G, DIM), jnp.float32)] * 3
        + [pltpu.SemaphoreType.DMA] * 2
    ),
)
def _sc_combine(ys_hbm, ia_hbm, ic_hbm, wa_hbm, wc_hbm, out_hbm,
                iav, icv, wav, wcv, bufa, bufc, bufo, sa, sc):
    wid = lax.axis_index("s") * 2 + lax.axis_index("c")
    pltpu.sync_copy(ia_hbm.at[wid], iav)
    pltpu.sync_copy(ic_hbm.at[wid], icv)
    pltpu.sync_copy(wa_hbm.at[wid], wav)
    pltpu.sync_copy(wc_hbm.at[wid], wcv)
    base = wid * TPW
    for c in range(NCH_G):
        ca = pltpu.async_copy(ys_hbm.at[iav.at[c]], bufa, sa)
        cc = pltpu.async_copy(ys_hbm.at[icv.at[c]], bufc, sc)
        ca.wait()
        cc.wait()

        def _row(t, _):
            w0b = wav[c * CH_G + t, pl.ds(0, 16)]
            w1b = wcv[c * CH_G + t, pl.ds(0, 16)]
            for k in range(DIM // 16):
                va = bufa[t, pl.ds(16 * k, 16)]
                vc = bufc[t, pl.ds(16 * k, 16)]
                bufo[t, pl.ds(16 * k, 16)] = w0b * va + w1b * vc
            return 0

        lax.fori_loop(0, CH_G, _row, 0)
        pltpu.sync_copy(bufo, out_hbm.at[pl.ds(base + c * CH_G, CH_G)])


def kernel(x, gate_w, W1, W2, W3):
    B, T, C = x.shape
    x_flat = x.reshape(N, C)

    pos0, pos1, w0, w1, cnt, offp, aux = _gate(x_flat, gate_w)
    tile_e, tile_m = _tile_maps(cnt[0], offp[0])
    rf, par, nxte, hn = _ffn_sched(tile_e)

    p0r = pos0.reshape(NW, NCH_S, CH_S)
    p1r = pos1.reshape(NW, NCH_S, CH_S)
    xs = _sc_dispatch(x_flat, p0r, p1r)

    ys = _ffn(xs, W1, W3, W2, tile_e, tile_m, rf, par, nxte, hn)

    ia = pos0.reshape(NW, NCH_G, CH_G)
    ic = pos1.reshape(NW, NCH_G, CH_G)
    wa = jnp.broadcast_to(w0.reshape(NW, TPW, 1), (NW, TPW, 16))
    wc = jnp.broadcast_to(w1.reshape(NW, TPW, 1), (NW, TPW, 16))
    out = _sc_combine(ys, ia, ic, wa, wc)
    return out.reshape(B, T, C), aux.reshape(())[()]


# speedup vs baseline: 1.0690x; 1.0184x over previous
import functools
import jax
import jax.numpy as jnp
from jax import lax
from jax.experimental import pallas as pl
from jax.experimental.pallas import tpu as pltpu
from jax.experimental.pallas import tpu_sc as plsc

DIM = 1024
HIDDEN = 2816
HHALF = HIDDEN // 2
N_EXPERTS = 8
AUX_WEIGHT = 0.01

N = 4096
BG = 512
BM = 256
NM = N // BG
SMAX = 9984
NT = SMAX // BM

NW = 32
TPW = N // NW
CH_S = 32
NCH_S = TPW // CH_S
CH_G = 16
NCH_G = TPW // CH_G


def _top2(logits, n):
    iota = lax.broadcasted_iota(jnp.int32, (n, N_EXPERTS), 1)
    m1 = jnp.max(logits, axis=1, keepdims=True)
    i1 = jnp.min(jnp.where(logits == m1, iota, N_EXPERTS), axis=1,
                 keepdims=True)
    l2 = jnp.where(iota == i1, -jnp.inf, logits)
    m2 = jnp.max(l2, axis=1, keepdims=True)
    i2 = jnp.min(jnp.where(l2 == m2, iota, N_EXPERTS), axis=1,
                 keepdims=True)
    oh1 = (iota == i1).astype(jnp.float32)
    oh2 = (iota == i2).astype(jnp.float32)
    return m1, m2, oh1, oh2


def _gate_a_kernel(x_ref, gw_ref, cnt_ref, offp_ref, aux_ref,
                   psum_s, fsum_s, cnt_s):
    m = pl.program_id(0)
    n = x_ref.shape[0]
    logits = jnp.dot(x_ref[...], gw_ref[...].T,
                     preferred_element_type=jnp.float32)
    m1, m2, oh1, oh2 = _top2(logits, n)

    pe = jnp.exp(logits - m1)
    probs = pe / jnp.sum(pe, axis=1, keepdims=True)
    psum = jnp.sum(probs, axis=0, keepdims=True)
    fsum = jnp.sum(oh1, axis=0, keepdims=True)
    csum = jnp.sum(oh1 + oh2, axis=0, keepdims=True)

    @pl.when(m == 0)
    def _init():
        psum_s[...] = psum
        fsum_s[...] = fsum
        cnt_s[...] = csum

    @pl.when(m != 0)
    def _acc():
        psum_s[...] += psum
        fsum_s[...] += fsum
        cnt_s[...] += csum

    @pl.when(m == NM - 1)
    def _fin():
        padded = jnp.floor((cnt_s[...] + (BM - 1)) / BM) * BM
        ei = lax.broadcasted_iota(jnp.int32, (N_EXPERTS, N_EXPERTS), 0)
        ej = lax.broadcasted_iota(jnp.int32, (N_EXPERTS, N_EXPERTS), 1)
        slt = (ei < ej).astype(jnp.float32)
        offp = jnp.dot(padded, slt, preferred_element_type=jnp.float32)
        cnt_ref[...] = cnt_s[...].astype(jnp.int32)
        offp_ref[...] = offp.astype(jnp.int32)
        ntok = jnp.float32(N)
        aux_ref[...] = (AUX_WEIGHT * N_EXPERTS
                        * jnp.sum(psum_s[...] * fsum_s[...],
                                  keepdims=True).reshape(1, 1)
                        / (ntok * ntok))


def _gate_b_kernel(x_ref, gw_ref, offp_ref,
                   pos0_ref, pos1_ref, w0_ref, w1_ref, carry_s):
    m = pl.program_id(0)
    n = x_ref.shape[0]
    logits = jnp.dot(x_ref[...], gw_ref[...].T,
                     preferred_element_type=jnp.float32)
    m1, m2, oh1, oh2 = _top2(logits, n)

    t = jnp.exp(m2 - m1)
    w0_ref[...] = 1.0 / (1.0 + t)
    w1_ref[...] = t / (1.0 + t)

    @pl.when(m == 0)
    def _init():
        carry_s[...] = jnp.zeros_like(carry_s)

    o_all = jnp.concatenate([oh1, oh2], axis=0)
    ri = lax.broadcasted_iota(jnp.int32, (2 * n, 2 * n), 0)
    rj = lax.broadcasted_iota(jnp.int32, (2 * n, 2 * n), 1)
    tri = (ri >= rj).astype(jnp.bfloat16)
    csum = jnp.dot(tri, o_all.astype(jnp.bfloat16),
                   preferred_element_type=jnp.float32)
    mtx = csum + carry_s[...] + offp_ref[...].astype(jnp.float32) - 1.0
    posall = jnp.sum(mtx * o_all, axis=1, keepdims=True)
    pos0_ref[...] = posall[:n].astype(jnp.int32)
    pos1_ref[...] = posall[n:].astype(jnp.int32)
    carry_s[...] += jnp.sum(o_all, axis=0, keepdims=True)


def _gate(x_flat, gate_w):
    cnt, offp, aux = pl.pallas_call(
        _gate_a_kernel,
        grid=(NM,),
        in_specs=[
            pl.BlockSpec((BG, DIM), lambda m: (m, 0)),
            pl.BlockSpec((N_EXPERTS, DIM), lambda m: (0, 0)),
        ],
        out_specs=[
            pl.BlockSpec((1, N_EXPERTS), lambda m: (0, 0)),
            pl.BlockSpec((1, N_EXPERTS), lambda m: (0, 0)),
            pl.BlockSpec((1, 1), lambda m: (0, 0)),
        ],
        out_shape=[
            jax.ShapeDtypeStruct((1, N_EXPERTS), jnp.int32),
            jax.ShapeDtypeStruct((1, N_EXPERTS), jnp.int32),
            jax.ShapeDtypeStruct((1, 1), jnp.float32),
        ],
        scratch_shapes=[pltpu.VMEM((1, N_EXPERTS), jnp.float32)] * 3,
    )(x_flat, gate_w)

    pos0, pos1, w0, w1 = pl.pallas_call(
        _gate_b_kernel,
        grid=(NM,),
        in_specs=[
            pl.BlockSpec((BG, DIM), lambda m: (m, 0)),
            pl.BlockSpec((N_EXPERTS, DIM), lambda m: (0, 0)),
            pl.BlockSpec((1, N_EXPERTS), lambda m: (0, 0)),
        ],
        out_specs=[
            pl.BlockSpec((BG, 1), lambda m: (m, 0)),
            pl.BlockSpec((BG, 1), lambda m: (m, 0)),
            pl.BlockSpec((BG, 1), lambda m: (m, 0)),
            pl.BlockSpec((BG, 1), lambda m: (m, 0)),
        ],
        out_shape=[
            jax.ShapeDtypeStruct((N, 1), jnp.int32),
            jax.ShapeDtypeStruct((N, 1), jnp.int32),
            jax.ShapeDtypeStruct((N, 1), jnp.float32),
            jax.ShapeDtypeStruct((N, 1), jnp.float32),
        ],
        scratch_shapes=[pltpu.VMEM((1, N_EXPERTS), jnp.float32)],
    )(x_flat, gate_w, offp)
    return pos0, pos1, w0, w1, cnt, offp, aux


_sc_mesh = plsc.VectorSubcoreMesh(core_axis_name="c", subcore_axis_name="s")


@functools.partial(
    pl.kernel,
    mesh=_sc_mesh,
    out_type=jax.ShapeDtypeStruct((SMAX, DIM), jnp.float32),
    scratch_types=[
        pltpu.VMEM((NCH_S, CH_S), jnp.int32),
        pltpu.VMEM((NCH_S, CH_S), jnp.int32),
        pltpu.VMEM((CH_S, DIM), jnp.float32),
        pltpu.SemaphoreType.DMA,
        pltpu.SemaphoreType.DMA,
    ],
)
def _sc_dispatch(x_hbm, pos0_hbm, pos1_hbm, xs_hbm,
                 p0_v, p1_v, xbuf, sem0, sem1):
    wid = lax.axis_index("s") * 2 + lax.axis_index("c")
    pltpu.sync_copy(pos0_hbm.at[wid], p0_v)
    pltpu.sync_copy(pos1_hbm.at[wid], p1_v)
    base = wid * TPW
    for c in range(NCH_S):
        pltpu.sync_copy(x_hbm.at[pl.ds(base + c * CH_S, CH_S)], xbuf)
        cp0 = pltpu.async_copy(xbuf, xs_hbm.at[p0_v.at[c]], sem0)
        cp1 = pltpu.async_copy(xbuf, xs_hbm.at[p1_v.at[c]], sem1)
        cp0.wait()
        cp1.wait()


def _ffn_kernel(h2, te_ref, tm_ref, rf_ref, par_ref, nxte_ref, hn_ref,
                xs_ref, w1_hbm, w3_hbm, w2_hbm, ysin_ref, ys_ref,
                w1b, w3b, w2b, sem1, sem3, sem2):
    i = pl.program_id(0)
    te = te_ref[i]
    p = par_ref[i]
    first = rf_ref[i] == 1

    def _w_copies(e, slot):
        return (
            pltpu.make_async_copy(
                w1_hbm.at[e, pl.ds(h2 * HHALF, HHALF), :],
                w1b.at[slot], sem1.at[slot]),
            pltpu.make_async_copy(
                w3_hbm.at[e, pl.ds(h2 * HHALF, HHALF), :],
                w3b.at[slot], sem3.at[slot]),
            pltpu.make_async_copy(
                w2_hbm.at[e, :, pl.ds(h2 * HHALF, HHALF)],
                w2b.at[slot], sem2.at[slot]),
        )

    def _start(e, slot):
        for c in _w_copies(e, slot):
            c.start()

    def _wait(e, slot):
        for c in _w_copies(e, slot):
            c.wait()

    @pl.when(i == 0)
    def _pro0():
        _start(te, 0)

    @pl.when((i == 0) & (hn_ref[i] == 1))
    def _pro1():
        _start(nxte_ref[i], 1)

    @pl.when(first & (i != 0) & (hn_ref[i] == 1) & (p == 0))
    def _fetch_n1():
        _start(nxte_ref[i], 1)

    @pl.when(first & (i != 0) & (hn_ref[i] == 1) & (p == 1))
    def _fetch_n0():
        _start(nxte_ref[i], 0)

    @pl.when(first & (p == 0))
    def _wait0():
        _wait(te, 0)

    @pl.when(first & (p == 1))
    def _wait1():
        _wait(te, 1)

    def _compute(slot):
        xl = xs_ref[...].astype(jnp.bfloat16)
        hb = (jax.nn.silu(jnp.dot(xl, w1b[slot].astype(jnp.bfloat16).T,
                                  preferred_element_type=jnp.float32))
              * jnp.dot(xl, w3b[slot].astype(jnp.bfloat16).T,
                        preferred_element_type=jnp.float32))
        contrib = jnp.dot(hb.astype(jnp.bfloat16),
                          w2b[slot].astype(jnp.bfloat16).T,
                          preferred_element_type=jnp.float32)
        if h2 == 0:
            ys_ref[...] = contrib
        else:
            ys_ref[...] = ysin_ref[...] + contrib

    @pl.when(p == 0)
    def _c0():
        _compute(0)

    @pl.when(p == 1)
    def _c1():
        _compute(1)


def _ffn(xs, W1, W3, W2, tile_e, tile_m, rf, par, nxte, hn):
    def half(h2, ysin):
        grid_spec = pltpu.PrefetchScalarGridSpec(
            num_scalar_prefetch=6,
            grid=(NT,),
            in_specs=[
                pl.BlockSpec((BM, DIM), lambda i, *pref: (pref[1][i], 0)),
                pl.BlockSpec(memory_space=pltpu.MemorySpace.HBM),
                pl.BlockSpec(memory_space=pltpu.MemorySpace.HBM),
                pl.BlockSpec(memory_space=pltpu.MemorySpace.HBM),
                (pl.BlockSpec((BM, DIM), lambda i, *pref: (pref[1][i], 0))
                 if h2 == 1 else
                 pl.BlockSpec((BM, DIM), lambda i, *pref: (0, 0))),
            ],
            out_specs=pl.BlockSpec((BM, DIM), lambda i, *pref: (pref[1][i], 0)),
            scratch_shapes=[
                pltpu.VMEM((2, HHALF, DIM), jnp.float32),
                pltpu.VMEM((2, HHALF, DIM), jnp.float32),
                pltpu.VMEM((2, DIM, HHALF), jnp.float32),
                pltpu.SemaphoreType.DMA((2,)),
                pltpu.SemaphoreType.DMA((2,)),
                pltpu.SemaphoreType.DMA((2,)),
            ],
        )
        return pl.pallas_call(
            functools.partial(_ffn_kernel, h2),
            grid_spec=grid_spec,
            out_shape=jax.ShapeDtypeStruct((SMAX, DIM), jnp.float32),
        )(tile_e, tile_m, rf, par, nxte, hn, xs, W1, W3, W2, ysin)

    ys0 = half(0, xs)
    return half(1, ys0)


def tile_m_idx(pref, i):
    return pref[1][i]


def _tile_maps(cnt, offp):
    t_e = (cnt + BM - 1) // BM
    st = offp // BM
    ends = st + t_e
    nta = jnp.sum(t_e)
    i = jnp.arange(NT, dtype=jnp.int32)
    e_i = jnp.sum((i[:, None] >= ends[None, :]).astype(jnp.int32), axis=1)
    e_last = jnp.max(jnp.where(cnt > 0, jnp.arange(N_EXPERTS, dtype=jnp.int32),
                               0))
    tile_e = jnp.where(i < nta, jnp.minimum(e_i, N_EXPERTS - 1), e_last)
    tile_m = jnp.where(i < nta, i, NT - 1)
    return tile_e.astype(jnp.int32), tile_m.astype(jnp.int32)


def _ffn_sched(te):
    rf = jnp.concatenate([jnp.ones((1,), jnp.int32),
                          (te[1:] != te[:-1]).astype(jnp.int32)])
    ridx = jnp.cumsum(rf) - 1
    par = (ridx % 2).astype(jnp.int32)
    r = jnp.arange(NT, dtype=jnp.int32)
    r_onehot = (ridx[None, :] == r[:, None])
    start_pos = jnp.argmax(r_onehot, axis=1).astype(jnp.int32)
    run_e = te[start_pos]
    nrun = ridx[-1] + 1
    has = ((r + 1) < nrun).astype(jnp.int32)
    nxt_run_e = jnp.where((r + 1) < nrun,
                          run_e[jnp.minimum(r + 1, NT - 1)], run_e)
    nxte = nxt_run_e[ridx].astype(jnp.int32)
    hn = has[ridx].astype(jnp.int32)
    return rf.astype(jnp.int32), par, nxte, hn


@functools.partial(
    pl.kernel,
    mesh=_sc_mesh,
    out_type=jax.ShapeDtypeStruct((N, DIM), jnp.float32),
    scratch_types=(
        [pltpu.VMEM((NCH_G, CH_G), jnp.int32)] * 2
        + [pltpu.VMEM((TPW, 16), jnp.float32)] * 2
        + [pltpu.VMEM((2, CH_G, DIM), jnp.float32)] * 2
        + [pltpu.VMEM((CH_G, DIM), jnp.float32)]
        + [pltpu.SemaphoreType.DMA((2,))] * 2
    ),
)
def _sc_combine(ys_hbm, ia_hbm, ic_hbm, wa_hbm, wc_hbm, out_hbm,
                iav, icv, wav, wcv, bufa, bufc, bufo, sa, sc):
    wid = lax.axis_index("s") * 2 + lax.axis_index("c")
    pltpu.sync_copy(ia_hbm.at[wid], iav)
    pltpu.sync_copy(ic_hbm.at[wid], icv)
    pltpu.sync_copy(wa_hbm.at[wid], wav)
    pltpu.sync_copy(wc_hbm.at[wid], wcv)
    base = wid * TPW

    def _gath(c):
        sl = c % 2
        return (pltpu.async_copy(ys_hbm.at[iav.at[c]], bufa.at[sl], sa.at[sl]),
                pltpu.async_copy(ys_hbm.at[icv.at[c]], bufc.at[sl], sc.at[sl]))

    _gath(0)
    for c in range(NCH_G):
        if c + 1 < NCH_G:
            _gath(c + 1)
        sl = c % 2
        pltpu.make_async_copy(ys_hbm.at[iav.at[c]], bufa.at[sl],
                              sa.at[sl]).wait()
        pltpu.make_async_copy(ys_hbm.at[icv.at[c]], bufc.at[sl],
                              sc.at[sl]).wait()

        def _row(t, _):
            w0b = wav[c * CH_G + t, pl.ds(0, 16)]
            w1b = wcv[c * CH_G + t, pl.ds(0, 16)]
            for k in range(DIM // 16):
                va = bufa[sl, t, pl.ds(16 * k, 16)]
                vc = bufc[sl, t, pl.ds(16 * k, 16)]
                bufo[t, pl.ds(16 * k, 16)] = w0b * va + w1b * vc
            return 0

        lax.fori_loop(0, CH_G, _row, 0)
        pltpu.sync_copy(bufo, out_hbm.at[pl.ds(base + c * CH_G, CH_G)])


def kernel(x, gate_w, W1, W2, W3):
    B, T, C = x.shape
    x_flat = x.reshape(N, C)

    pos0, pos1, w0, w1, cnt, offp, aux = _gate(x_flat, gate_w)
    tile_e, tile_m = _tile_maps(cnt[0], offp[0])
    rf, par, nxte, hn = _ffn_sched(tile_e)

    p0r = pos0.reshape(NW, NCH_S, CH_S)
    p1r = pos1.reshape(NW, NCH_S, CH_S)
    xs = _sc_dispatch(x_flat, p0r, p1r)

    ys = _ffn(xs, W1, W3, W2, tile_e, tile_m, rf, par, nxte, hn)

    ia = pos0.reshape(NW, NCH_G, CH_G)
    ic = pos1.reshape(NW, NCH_G, CH_G)
    wa = jnp.broadcast_to(w0.reshape(NW, TPW, 1), (NW, TPW, 16))
    wc = jnp.broadcast_to(w1.reshape(NW, TPW, 1), (NW, TPW, 16))
    out = _sc_combine(ys, ia, ic, wa, wc)
    return out.reshape(B, T, C), aux.reshape(())[()]
